# baseline (speedup 1.0000x reference)
import functools

import jax
import jax.numpy as jnp
from jax import lax
from jax.experimental import pallas as pl
from jax.experimental.pallas import tpu as pltpu
from jax.experimental.pallas import tpu_sc as plsc

_TN = 512
_TK = 512


def _argmin_body(z_ref, cb_ref, idx_ref):
    tn = z_ref.shape[0]
    ktot = cb_ref.shape[0]
    z = z_ref[...]
    z_sq = jnp.sum(z * z, axis=1, keepdims=True)

    def step(k, carry):
        best_val, best_idx = carry
        cb = cb_ref[pl.ds(k * _TK, _TK), :]
        c_sq = jnp.sum(cb * cb, axis=1)
        mm = lax.dot_general(z, cb, (((1,), (1,)), ((), ())),
                             preferred_element_type=jnp.float32)
        dists = (z_sq + c_sq[None, :]) - 2.0 * mm
        cols = lax.broadcasted_iota(jnp.int32, (tn, _TK), 1) + k * _TK
        tile_min = jnp.min(dists, axis=1, keepdims=True)
        tile_idx = jnp.min(jnp.where(dists == tile_min, cols, ktot),
                           axis=1, keepdims=True)
        better = tile_min < best_val
        return (jnp.where(better, tile_min, best_val),
                jnp.where(better, tile_idx, best_idx))

    init = (jnp.full((tn, 1), jnp.inf, jnp.float32),
            jnp.zeros((tn, 1), jnp.int32))
    _, best_idx = lax.fori_loop(0, ktot // _TK, step, init)
    idx_ref[...] = best_idx


def _argmin_tc(z, codebook):
    n, d = z.shape
    k = codebook.shape[0]
    out = pl.pallas_call(
        _argmin_body,
        grid=(n // _TN,),
        in_specs=[pl.BlockSpec((_TN, d), lambda i: (i, 0)),
                  pl.BlockSpec((k, d), lambda i: (0, 0))],
        out_specs=pl.BlockSpec((_TN, 1), lambda i: (i, 0)),
        out_shape=jax.ShapeDtypeStruct((n, 1), jnp.int32),
    )(z, codebook)
    return out.reshape(n)


def _gather_sc(codebook, idx):
    n = idx.shape[0]
    d = codebook.shape[1]
    info = plsc.get_sparse_core_info()
    nw = info.num_cores * info.num_subcores
    b_per_w = n // nw
    chunk = 128
    n_chunks = b_per_w // chunk
    mesh = plsc.VectorSubcoreMesh(core_axis_name="c", subcore_axis_name="s")

    @functools.partial(
        pl.kernel, mesh=mesh,
        out_type=jax.ShapeDtypeStruct((n, d), jnp.float32),
        scratch_types=[
            pltpu.VMEM((chunk,), jnp.int32),
            pltpu.VMEM((chunk, d), jnp.float32),
            pltpu.SemaphoreType.DMA,
        ],
    )
    def k(table_hbm, idx_hbm, out_hbm, idx_v, rows_v, sem):
        wid = lax.axis_index("s") * info.num_cores + lax.axis_index("c")
        base = wid * b_per_w

        def body(c, carry):
            off = base + c * chunk
            pltpu.sync_copy(idx_hbm.at[pl.ds(off, chunk)], idx_v)
            pltpu.async_copy(table_hbm.at[idx_v], rows_v, sem).wait()
            pltpu.sync_copy(rows_v, out_hbm.at[pl.ds(off, chunk)])
            return carry

        lax.fori_loop(0, n_chunks, body, 0)

    return k(codebook, idx)


def kernel(z, codebook):
    indices = _argmin_tc(z, codebook)
    z_q = _gather_sc(codebook, indices)
    return (z_q, indices)


# speedup vs baseline: 1.0219x; 1.0219x over previous
import functools

import jax
import jax.numpy as jnp
from jax import lax
from jax.experimental import pallas as pl
from jax.experimental.pallas import tpu as pltpu
from jax.experimental.pallas import tpu_sc as plsc

_TN = 512
_TK = 512


def _argmin_body(z_ref, cb_ref, csq_ref, idx_ref):
    tn = z_ref.shape[0]
    ktot = cb_ref.shape[0]
    z = z_ref[...]
    z_sq = jnp.sum(z * z, axis=1, keepdims=True)
    zm2 = z * (-2.0)
    cols = lax.broadcasted_iota(jnp.int32, (tn, _TK), 1)

    def step(k, carry):
        best_val, best_idx = carry
        cb = cb_ref[pl.ds(k * _TK, _TK), :]
        c_sq = csq_ref[:, pl.ds(k * _TK, _TK)]
        mm2 = lax.dot_general(zm2, cb, (((1,), (1,)), ((), ())),
                              preferred_element_type=jnp.float32)
        dists = (z_sq + c_sq) + mm2
        tile_min = jnp.min(dists, axis=1, keepdims=True)
        tile_idx = jnp.min(jnp.where(dists == tile_min, cols, _TK),
                           axis=1, keepdims=True) + k * _TK
        better = tile_min < best_val
        return (jnp.where(better, tile_min, best_val),
                jnp.where(better, tile_idx, best_idx))

    init = (jnp.full((tn, 1), jnp.inf, jnp.float32),
            jnp.zeros((tn, 1), jnp.int32))
    _, best_idx = lax.fori_loop(0, ktot // _TK, step, init)
    idx_ref[...] = best_idx


def _argmin_tc(z, codebook):
    n, d = z.shape
    k = codebook.shape[0]
    c_sq = jnp.sum(codebook * codebook, axis=-1)[None, :]
    out = pl.pallas_call(
        _argmin_body,
        grid=(n // _TN,),
        in_specs=[pl.BlockSpec((_TN, d), lambda i: (i, 0)),
                  pl.BlockSpec((k, d), lambda i: (0, 0)),
                  pl.BlockSpec((1, k), lambda i: (0, 0))],
        out_specs=pl.BlockSpec((_TN, 1), lambda i: (i, 0)),
        out_shape=jax.ShapeDtypeStruct((n, 1), jnp.int32),
    )(z, codebook, c_sq)
    return out.reshape(n)


def _gather_sc(codebook, idx):
    n = idx.shape[0]
    d = codebook.shape[1]
    info = plsc.get_sparse_core_info()
    nw = info.num_cores * info.num_subcores
    b_per_w = n // nw
    chunk = 128
    n_chunks = b_per_w // chunk
    mesh = plsc.VectorSubcoreMesh(core_axis_name="c", subcore_axis_name="s")

    @functools.partial(
        pl.kernel, mesh=mesh,
        out_type=jax.ShapeDtypeStruct((n, d), jnp.float32),
        scratch_types=[
            pltpu.VMEM((chunk,), jnp.int32),
            pltpu.VMEM((chunk, d), jnp.float32),
            pltpu.SemaphoreType.DMA,
        ],
    )
    def k(table_hbm, idx_hbm, out_hbm, idx_v, rows_v, sem):
        wid = lax.axis_index("s") * info.num_cores + lax.axis_index("c")
        base = wid * b_per_w

        def body(c, carry):
            off = base + c * chunk
            pltpu.sync_copy(idx_hbm.at[pl.ds(off, chunk)], idx_v)
            pltpu.async_copy(table_hbm.at[idx_v], rows_v, sem).wait()
            pltpu.sync_copy(rows_v, out_hbm.at[pl.ds(off, chunk)])
            return carry

        lax.fori_loop(0, n_chunks, body, 0)

    return k(codebook, idx)


def kernel(z, codebook):
    indices = _argmin_tc(z, codebook)
    z_q = _gather_sc(codebook, indices)
    return (z_q, indices)


# speedup vs baseline: 1.3550x; 1.3260x over previous
import functools

import jax
import jax.numpy as jnp
from jax import lax
from jax.experimental import pallas as pl
from jax.experimental.pallas import tpu as pltpu
from jax.experimental.pallas import tpu_sc as plsc

_TN = 512
_TK = 512


_NL = 128


def _argmin_body(z_ref, cb_ref, csq_ref, idx_ref, a_ref, rv_ref, ri_ref):
    tn = z_ref.shape[0]
    ktot = cb_ref.shape[0]
    nsub = _TK // _NL
    z = z_ref[...]
    z_sq = jnp.sum(z * z, axis=1, keepdims=True)
    zm2 = z * (-2.0)
    a_ref[...] = z_sq + csq_ref[...]
    rv_ref[...] = jnp.full((tn, _NL), jnp.inf, jnp.float32)
    ri_ref[...] = jnp.zeros((tn, _NL), jnp.int32)

    def step(k, c):
        cb = cb_ref[pl.ds(k * _TK, _TK), :]
        mm2 = lax.dot_general(zm2, cb, (((1,), (1,)), ((), ())),
                              preferred_element_type=jnp.float32)
        dists = a_ref[:, pl.ds(k * _TK, _TK)] + mm2
        rv, ri = rv_ref[...], ri_ref[...]
        for sub in range(nsub):
            d = dists[:, sub * _NL:(sub + 1) * _NL]
            s = k * nsub + sub
            take = d < rv
            rv = jnp.minimum(d, rv)
            ri = jnp.where(take, s, ri)
        rv_ref[...], ri_ref[...] = rv, ri
        return c

    lax.fori_loop(0, ktot // _TK, step, 0)
    rv, ri = rv_ref[...], ri_ref[...]
    lane = lax.broadcasted_iota(jnp.int32, (tn, _NL), 1)
    gcol = ri * _NL + lane
    m = jnp.min(rv, axis=1, keepdims=True)
    idx_ref[...] = jnp.min(jnp.where(rv == m, gcol, ktot),
                           axis=1, keepdims=True)


def _argmin_tc(z, codebook):
    n, d = z.shape
    k = codebook.shape[0]
    c_sq = jnp.sum(codebook * codebook, axis=-1)[None, :]
    out = pl.pallas_call(
        _argmin_body,
        grid=(n // _TN,),
        in_specs=[pl.BlockSpec((_TN, d), lambda i: (i, 0)),
                  pl.BlockSpec((k, d), lambda i: (0, 0)),
                  pl.BlockSpec((1, k), lambda i: (0, 0))],
        out_specs=pl.BlockSpec((_TN, 1), lambda i: (i, 0)),
        out_shape=jax.ShapeDtypeStruct((n, 1), jnp.int32),
        scratch_shapes=[
            pltpu.VMEM((_TN, k), jnp.float32),
            pltpu.VMEM((_TN, _NL), jnp.float32),
            pltpu.VMEM((_TN, _NL), jnp.int32),
        ],
    )(z, codebook, c_sq)
    return out.reshape(n)


def _gather_sc(codebook, idx):
    n = idx.shape[0]
    d = codebook.shape[1]
    info = plsc.get_sparse_core_info()
    nw = info.num_cores * info.num_subcores
    b_per_w = n // nw
    chunk = 128
    n_chunks = b_per_w // chunk
    mesh = plsc.VectorSubcoreMesh(core_axis_name="c", subcore_axis_name="s")

    @functools.partial(
        pl.kernel, mesh=mesh,
        out_type=jax.ShapeDtypeStruct((n, d), jnp.float32),
        scratch_types=[
            pltpu.VMEM((chunk,), jnp.int32),
            pltpu.VMEM((chunk, d), jnp.float32),
            pltpu.SemaphoreType.DMA,
        ],
    )
    def k(table_hbm, idx_hbm, out_hbm, idx_v, rows_v, sem):
        wid = lax.axis_index("s") * info.num_cores + lax.axis_index("c")
        base = wid * b_per_w

        def body(c, carry):
            off = base + c * chunk
            pltpu.sync_copy(idx_hbm.at[pl.ds(off, chunk)], idx_v)
            pltpu.async_copy(table_hbm.at[idx_v], rows_v, sem).wait()
            pltpu.sync_copy(rows_v, out_hbm.at[pl.ds(off, chunk)])
            return carry

        lax.fori_loop(0, n_chunks, body, 0)

    return k(codebook, idx)


def kernel(z, codebook):
    indices = _argmin_tc(z, codebook)
    z_q = _gather_sc(codebook, indices)
    return (z_q, indices)


# speedup vs baseline: 2.4617x; 1.8167x over previous
import functools

import jax
import jax.numpy as jnp
from jax import lax
from jax.experimental import pallas as pl
from jax.experimental.pallas import tpu as pltpu
from jax.experimental.pallas import tpu_sc as plsc

_TN = 512
_TK = 512


_NL = 128


def _argmin_body(z_ref, cb_ref, csq_ref, idx_ref, a_ref):
    tn = z_ref.shape[0]
    ktot = cb_ref.shape[0]
    nsub = _TK // _NL
    z = z_ref[...]
    z_sq = jnp.sum(z * z, axis=1, keepdims=True)
    zm2 = z * (-2.0)
    a_ref[...] = z_sq + csq_ref[...]
    rv = jnp.full((tn, _NL), jnp.inf, jnp.float32)
    ri = jnp.zeros((tn, _NL), jnp.int32)

    for k in range(ktot // _TK):
        cb = cb_ref[pl.ds(k * _TK, _TK), :]
        mm2 = lax.dot_general(zm2, cb, (((1,), (1,)), ((), ())),
                              preferred_element_type=jnp.float32)
        dists = a_ref[:, pl.ds(k * _TK, _TK)] + mm2
        for sub in range(nsub):
            d = dists[:, sub * _NL:(sub + 1) * _NL]
            s = k * nsub + sub
            take = d < rv
            rv = jnp.minimum(d, rv)
            ri = jnp.where(take, s, ri)
    lane = lax.broadcasted_iota(jnp.int32, (tn, _NL), 1)
    gcol = ri * _NL + lane
    m = jnp.min(rv, axis=1, keepdims=True)
    idx_ref[...] = jnp.min(jnp.where(rv == m, gcol, ktot),
                           axis=1, keepdims=True)


def _argmin_tc(z, codebook):
    n, d = z.shape
    k = codebook.shape[0]
    c_sq = jnp.sum(codebook * codebook, axis=-1)[None, :]
    out = pl.pallas_call(
        _argmin_body,
        grid=(n // _TN,),
        in_specs=[pl.BlockSpec((_TN, d), lambda i: (i, 0)),
                  pl.BlockSpec((k, d), lambda i: (0, 0)),
                  pl.BlockSpec((1, k), lambda i: (0, 0))],
        out_specs=pl.BlockSpec((_TN, 1), lambda i: (i, 0)),
        out_shape=jax.ShapeDtypeStruct((n, 1), jnp.int32),
        scratch_shapes=[
            pltpu.VMEM((_TN, k), jnp.float32),
        ],
    )(z, codebook, c_sq)
    return out.reshape(n)


def _gather_sc(codebook, idx):
    n = idx.shape[0]
    d = codebook.shape[1]
    info = plsc.get_sparse_core_info()
    nw = info.num_cores * info.num_subcores
    b_per_w = n // nw
    chunk = 128
    n_chunks = b_per_w // chunk
    mesh = plsc.VectorSubcoreMesh(core_axis_name="c", subcore_axis_name="s")

    @functools.partial(
        pl.kernel, mesh=mesh,
        out_type=jax.ShapeDtypeStruct((n, d), jnp.float32),
        scratch_types=[
            pltpu.VMEM((chunk,), jnp.int32),
            pltpu.VMEM((chunk, d), jnp.float32),
            pltpu.SemaphoreType.DMA,
        ],
    )
    def k(table_hbm, idx_hbm, out_hbm, idx_v, rows_v, sem):
        wid = lax.axis_index("s") * info.num_cores + lax.axis_index("c")
        base = wid * b_per_w

        def body(c, carry):
            off = base + c * chunk
            pltpu.sync_copy(idx_hbm.at[pl.ds(off, chunk)], idx_v)
            pltpu.async_copy(table_hbm.at[idx_v], rows_v, sem).wait()
            pltpu.sync_copy(rows_v, out_hbm.at[pl.ds(off, chunk)])
            return carry

        lax.fori_loop(0, n_chunks, body, 0)

    return k(codebook, idx)


def kernel(z, codebook):
    indices = _argmin_tc(z, codebook)
    z_q = _gather_sc(codebook, indices)
    return (z_q, indices)


# speedup vs baseline: 2.5092x; 1.0193x over previous
import functools

import jax
import jax.numpy as jnp
from jax import lax
from jax.experimental import pallas as pl
from jax.experimental.pallas import tpu as pltpu
from jax.experimental.pallas import tpu_sc as plsc

_TN = 1024
_TK = 512


_NL = 128


def _argmin_body(z_ref, cb_ref, csq_ref, idx_ref, a_ref):
    tn = z_ref.shape[0]
    ktot = cb_ref.shape[0]
    nsub = _TK // _NL
    z = z_ref[...]
    z_sq = jnp.sum(z * z, axis=1, keepdims=True)
    zm2 = z * (-2.0)
    a_ref[...] = z_sq + csq_ref[...]
    rv = jnp.full((tn, _NL), jnp.inf, jnp.float32)
    ri = jnp.zeros((tn, _NL), jnp.int32)

    for k in range(ktot // _TK):
        cb = cb_ref[pl.ds(k * _TK, _TK), :]
        mm2 = lax.dot_general(zm2, cb, (((1,), (1,)), ((), ())),
                              preferred_element_type=jnp.float32)
        dists = a_ref[:, pl.ds(k * _TK, _TK)] + mm2
        for sub in range(nsub):
            d = dists[:, sub * _NL:(sub + 1) * _NL]
            s = k * nsub + sub
            take = d < rv
            rv = jnp.minimum(d, rv)
            ri = jnp.where(take, s, ri)
    lane = lax.broadcasted_iota(jnp.int32, (tn, _NL), 1)
    gcol = ri * _NL + lane
    m = jnp.min(rv, axis=1, keepdims=True)
    idx_ref[...] = jnp.min(jnp.where(rv == m, gcol, ktot),
                           axis=1, keepdims=True)


def _argmin_tc(z, codebook):
    n, d = z.shape
    k = codebook.shape[0]
    c_sq = jnp.sum(codebook * codebook, axis=-1)[None, :]
    out = pl.pallas_call(
        _argmin_body,
        grid=(n // _TN,),
        in_specs=[pl.BlockSpec((_TN, d), lambda i: (i, 0)),
                  pl.BlockSpec((k, d), lambda i: (0, 0)),
                  pl.BlockSpec((1, k), lambda i: (0, 0))],
        out_specs=pl.BlockSpec((_TN, 1), lambda i: (i, 0)),
        out_shape=jax.ShapeDtypeStruct((n, 1), jnp.int32),
        scratch_shapes=[
            pltpu.VMEM((_TN, k), jnp.float32),
        ],
    )(z, codebook, c_sq)
    return out.reshape(n)


def _gather_sc(codebook, idx):
    n = idx.shape[0]
    d = codebook.shape[1]
    info = plsc.get_sparse_core_info()
    nw = info.num_cores * info.num_subcores
    b_per_w = n // nw
    chunk = 128
    n_chunks = b_per_w // chunk
    mesh = plsc.VectorSubcoreMesh(core_axis_name="c", subcore_axis_name="s")

    @functools.partial(
        pl.kernel, mesh=mesh,
        out_type=jax.ShapeDtypeStruct((n, d), jnp.float32),
        scratch_types=[
            pltpu.VMEM((chunk,), jnp.int32),
            pltpu.VMEM((chunk, d), jnp.float32),
            pltpu.SemaphoreType.DMA,
        ],
    )
    def k(table_hbm, idx_hbm, out_hbm, idx_v, rows_v, sem):
        wid = lax.axis_index("s") * info.num_cores + lax.axis_index("c")
        base = wid * b_per_w

        def body(c, carry):
            off = base + c * chunk
            pltpu.sync_copy(idx_hbm.at[pl.ds(off, chunk)], idx_v)
            pltpu.async_copy(table_hbm.at[idx_v], rows_v, sem).wait()
            pltpu.sync_copy(rows_v, out_hbm.at[pl.ds(off, chunk)])
            return carry

        lax.fori_loop(0, n_chunks, body, 0)

    return k(codebook, idx)


def kernel(z, codebook):
    indices = _argmin_tc(z, codebook)
    z_q = _gather_sc(codebook, indices)
    return (z_q, indices)
